# Initial kernel scaffold; baseline (speedup 1.0000x reference)
import jax
import jax.numpy as jnp
from jax.experimental import pallas as pl


def kernel(x, edge_index, W1, b1, W2, b2, W3, b3, W4, b4):
    raise NotImplementedError("write your pallas kernel here")


# speedup vs baseline: 38.0637x; 38.0637x over previous
import functools

import jax
import jax.numpy as jnp
from jax import lax
from jax.experimental import pallas as pl
from jax.experimental.pallas import tpu as pltpu
from jax.experimental.pallas import tpu_sc as plsc

_N = 100000
_E = 3200000
_H = 16

_NC = 2
_NS = 16
_NW = _NC * _NS
_GRP = 128
_GPD = 8
_CHUNK = _GRP * _GPD
_EPT = 100352
_EPAD = _EPT * _NW
_KITER = _EPT // _CHUNK
_NACC = 100096
_ZPT = _NACC // _NS

_mesh = plsc.VectorSubcoreMesh(core_axis_name="c", subcore_axis_name="s")
_sc_params = pltpu.CompilerParams(use_tc_tiling_on_sc=False)


def _zero_acc(rows, acc, s):
    @pl.loop(0, _CHUNK)
    def _(i):
        rows[i, :] = jnp.zeros((16,), jnp.float32)

    base = s * _ZPT
    nfull = _ZPT // _CHUNK
    for z in range(nfull):
        pltpu.sync_copy(rows, acc.at[pl.ds(base + z * _CHUNK, _CHUNK)])
    rem = _ZPT - nfull * _CHUNK
    if rem:
        pltpu.sync_copy(rows.at[pl.ds(0, rem)],
                        acc.at[pl.ds(base + nfull * _CHUNK, rem)])


def _dump_acc(acc, p0_hbm, p1_hbm, c, s):
    base = s * _ZPT

    @pl.when(c == 0)
    def _():
        pltpu.sync_copy(acc.at[pl.ds(base, _ZPT)], p0_hbm.at[pl.ds(base, _ZPT)])

    @pl.when(c == 1)
    def _():
        pltpu.sync_copy(acc.at[pl.ds(base, _ZPT)], p1_hbm.at[pl.ds(base, _ZPT)])


def _gather_scatter_body(table_hbm, src_hbm, dst_hbm, p0_hbm, p1_hbm,
                         srcv, dstv, rows, acc, gsem):
    c = lax.axis_index("c")
    s = lax.axis_index("s")
    wid = c * _NS + s

    _zero_acc(rows, acc, s)
    plsc.subcore_barrier()

    gpt = _EPT // _GRP

    @pl.loop(0, _KITER)
    def _(k):
        row0 = wid * gpt + k * _GPD
        pltpu.sync_copy(src_hbm.at[pl.ds(row0, _GPD)], srcv)
        pltpu.sync_copy(dst_hbm.at[pl.ds(row0, _GPD)], dstv)
        copies = [
            pltpu.async_copy(table_hbm.at[srcv.at[j]],
                             rows.at[pl.ds(j * _GRP, _GRP)], gsem)
            for j in range(_GPD)
        ]
        for cp in copies:
            cp.wait()
        for j in range(_GPD):
            pltpu.sync_copy(rows.at[pl.ds(j * _GRP, _GRP)],
                            acc.at[dstv.at[j]], add=True)

    plsc.subcore_barrier()
    _dump_acc(acc, p0_hbm, p1_hbm, c, s)


def _degree_body(dst_hbm, p0_hbm, p1_hbm, dstv, rows, acc):
    c = lax.axis_index("c")
    s = lax.axis_index("s")
    wid = c * _NS + s

    _zero_acc(rows, acc, s)

    @pl.loop(0, _GRP)
    def _(i):
        rows[i, :] = jnp.ones((16,), jnp.float32)

    plsc.subcore_barrier()

    gpt = _EPT // _GRP

    @pl.loop(0, _KITER)
    def _(k):
        row0 = wid * gpt + k * _GPD
        pltpu.sync_copy(dst_hbm.at[pl.ds(row0, _GPD)], dstv)
        for j in range(_GPD):
            pltpu.sync_copy(rows.at[pl.ds(0, _GRP)],
                            acc.at[dstv.at[j]], add=True)

    plsc.subcore_barrier()
    _dump_acc(acc, p0_hbm, p1_hbm, c, s)


@jax.jit
def _sc_edge_pass(table, src2d, dst2d):
    out = jax.ShapeDtypeStruct((_NACC, _H), jnp.float32)
    f = pl.kernel(
        _gather_scatter_body,
        out_type=[out, out],
        mesh=_mesh,
        scratch_types=[
            pltpu.VMEM((_GPD, _GRP), jnp.int32),
            pltpu.VMEM((_GPD, _GRP), jnp.int32),
            pltpu.VMEM((_CHUNK, _H), jnp.float32),
            pltpu.VMEM_SHARED((_NACC, _H), jnp.float32),
            pltpu.SemaphoreType.DMA,
        ],
        compiler_params=_sc_params,
    )
    return f(table, src2d, dst2d)


@jax.jit
def _sc_degree_pass(dst2d):
    out = jax.ShapeDtypeStruct((_NACC, _H), jnp.float32)
    f = pl.kernel(
        _degree_body,
        out_type=[out, out],
        mesh=_mesh,
        scratch_types=[
            pltpu.VMEM((_GPD, _GRP), jnp.int32),
            pltpu.VMEM((_CHUNK, _H), jnp.float32),
            pltpu.VMEM_SHARED((_NACC, _H), jnp.float32),
        ],
        compiler_params=_sc_params,
    )
    return f(dst2d)


_RB = 4000
_NBLK = _N // _RB

_blk = lambda: pl.BlockSpec((_RB, _H), lambda i: (i, 0))
_wblk = lambda: pl.BlockSpec((_H, _H), lambda i: (0, 0))
_bblk = lambda: pl.BlockSpec((1, _H), lambda i: (0, 0))
_o16 = jax.ShapeDtypeStruct((_N, _H), jnp.float32)


def _k_pre(pd0, pd1, x16, w1, dinv_ref, t1_ref):
    deg = pd0[...] + pd1[...] + 1.0
    dinv = lax.rsqrt(deg)
    dinv_ref[...] = dinv
    t1_ref[...] = dinv * jnp.dot(x16[...], w1[...],
                                 preferred_element_type=jnp.float32)


def _k_layer1(p0, p1, t1, dinv, w2, b1, h1_ref, t2_ref):
    d = dinv[...]
    h1 = jnp.maximum(d * (p0[...] + p1[...] + t1[...]) + b1[...], 0.0)
    h1_ref[...] = h1
    t2_ref[...] = d * jnp.dot(h1, w2[...], preferred_element_type=jnp.float32)


def _k_layer2(p0, p1, t2, dinv, w3, b2, t3_ref):
    d = dinv[...]
    h2 = d * (p0[...] + p1[...] + t2[...]) + b2[...]
    t3_ref[...] = d * jnp.dot(h2, w3[...], preferred_element_type=jnp.float32)


def _k_layer3(p0, p1, t3, dinv, h1, b3, t4_ref):
    d = dinv[...]
    h3 = jnp.maximum(d * (p0[...] + p1[...] + t3[...]) + b3[...] + h1[...], 0.0)
    t4_ref[...] = d * h3


def _k_layer4(p0, p1, t4, dinv, w4, b4, out_ref):
    z = dinv[...] * (p0[...] + p1[...] + t4[...])
    out_ref[...] = jnp.dot(z, w4[...],
                           preferred_element_type=jnp.float32) + b4[...]


def _tc_call(body, ins, in_specs, n_out):
    return pl.pallas_call(
        body,
        grid=(_NBLK,),
        in_specs=in_specs,
        out_specs=[_blk() for _ in range(n_out)],
        out_shape=[_o16] * n_out,
    )(*ins)


def kernel(x, edge_index, W1, b1, W2, b2, W3, b3, W4, b4):
    f32 = jnp.float32
    src = edge_index[0]
    dst = edge_index[1]
    pad = _EPAD - _E
    src2d = jnp.concatenate([src, jnp.zeros((pad,), jnp.int32)]).reshape(-1, _GRP)
    dst2d = jnp.concatenate([dst, jnp.full((pad,), _N, jnp.int32)]).reshape(-1, _GRP)

    x16 = jnp.pad(x, ((0, 0), (0, _H - x.shape[1])))
    w1p = jnp.pad(W1, ((0, _H - W1.shape[0]), (0, 0)))
    w4p = jnp.pad(W4, ((0, 0), (0, _H - W4.shape[1])))
    b4p = jnp.pad(b4, ((0, _H - b4.shape[0]),)).reshape(1, _H)
    b1r = b1.reshape(1, _H)
    b2r = b2.reshape(1, _H)
    b3r = b3.reshape(1, _H)

    pd0, pd1 = _sc_degree_pass(dst2d)

    dinv16, t1 = _tc_call(
        _k_pre, (pd0, pd1, x16, w1p),
        [_blk(), _blk(), _blk(), _wblk()], 2)

    p0, p1 = _sc_edge_pass(t1, src2d, dst2d)
    h1, t2 = _tc_call(
        _k_layer1, (p0, p1, t1, dinv16, W2, b1r),
        [_blk(), _blk(), _blk(), _blk(), _wblk(), _bblk()], 2)

    p0, p1 = _sc_edge_pass(t2, src2d, dst2d)
    (t3,) = _tc_call(
        _k_layer2, (p0, p1, t2, dinv16, W3, b2r),
        [_blk(), _blk(), _blk(), _blk(), _wblk(), _bblk()], 1)

    p0, p1 = _sc_edge_pass(t3, src2d, dst2d)
    (t4,) = _tc_call(
        _k_layer3, (p0, p1, t3, dinv16, h1, b3r),
        [_blk(), _blk(), _blk(), _blk(), _blk(), _bblk()], 1)

    p0, p1 = _sc_edge_pass(t4, src2d, dst2d)
    (out16,) = _tc_call(
        _k_layer4, (p0, p1, t4, dinv16, w4p, b4p),
        [_blk(), _blk(), _blk(), _blk(), _wblk(), _bblk()], 1)

    return out16[:, : W4.shape[1]].astype(f32)
